# baseline (speedup 1.0000x reference)
import functools

import jax
import jax.numpy as jnp
from jax import lax
from jax.experimental import pallas as pl
from jax.experimental.pallas import tpu as pltpu
from jax.experimental.pallas import tpu_sc as plsc

N = 10000
E = 320000
IN_DIM = 128
HID = 128
LAT = 64
CP = 64

NC = 2
NS = 16
NW = NC * NS
EPW = E // NW
K = 128
CH = EPW // K if EPW % K == 0 else EPW // K + 1
CH = CH + (CH % 2)
NR = ((N + 1) + NS * 8 - 1) // (NS * 8) * (NS * 8)
RPT = NR // NS

_mesh = plsc.VectorSubcoreMesh(core_axis_name="c", subcore_axis_name="s")


NBK = 48


@functools.partial(
    pl.kernel,
    out_type=[
        jax.ShapeDtypeStruct((NW, NR), jnp.float32),
        jax.ShapeDtypeStruct((NW, CH, K), jnp.int32),
        jax.ShapeDtypeStruct((NW, CH, K), jnp.int32),
    ],
    mesh=_mesh,
    scratch_types=[
        pltpu.VMEM((CH, K), jnp.int32),
        pltpu.VMEM((CH, K), jnp.int32),
        pltpu.VMEM((CH + 1, K), jnp.int32),
        pltpu.VMEM((CH + 1, K), jnp.int32),
        pltpu.VMEM((NR,), jnp.float32),
        pltpu.VMEM((NBK,), jnp.int32),
    ],
    compiler_params=pltpu.CompilerParams(needs_layout_passes=False),
)
def _deg_kernel(srcw_hbm, dstw_hbm, zdeg_hbm, degout, srcout, dstout,
                srcv, dstv, src2, dst2, degv, cnt):
    w = lax.axis_index("c") * NS + lax.axis_index("s")
    pltpu.sync_copy(srcw_hbm.at[w], srcv)
    pltpu.sync_copy(dstw_hbm.at[w], dstv)
    pltpu.sync_copy(zdeg_hbm, degv)
    ones = jnp.ones((16,), jnp.float32)
    onesi = jnp.ones((16,), jnp.int32)
    zi = jnp.zeros((16,), jnp.int32)
    ni = jnp.full((16,), N, jnp.int32)
    for b in range(NBK // 16):
        cnt[pl.ds(b * 16, 16)] = zi

    def hist_body(c, carry):
        for j in range(K // 16):
            src2[c, pl.ds(j * 16, 16)] = zi
            dst2[c, pl.ds(j * 16, 16)] = ni
            s = srcv[c, pl.ds(j * 16, 16)]
            d = dstv[c, pl.ds(j * 16, 16)]
            plsc.addupdate_scatter(degv, [d], ones)
            plsc.addupdate_scatter(cnt, [jnp.right_shift(s, 8)], onesi)
        return carry

    lax.fori_loop(0, CH, hist_body, 0)
    for j in range(K // 16):
        src2[CH, pl.ds(j * 16, 16)] = zi
        dst2[CH, pl.ds(j * 16, 16)] = ni
    pltpu.sync_copy(degv, degout.at[w])

    h0 = cnt[pl.ds(0, 16)]
    h1 = cnt[pl.ds(16, 16)]
    h2 = cnt[pl.ds(32, 16)]
    t0 = jnp.sum(h0)
    t1 = jnp.sum(h1)
    cnt[pl.ds(0, 16)] = plsc.cumsum(h0) - h0
    cnt[pl.ds(16, 16)] = plsc.cumsum(h1) - h1 + t0
    cnt[pl.ds(32, 16)] = plsc.cumsum(h2) - h2 + (t0 + t1)

    conv = jnp.min(plsc.scan_count(zi)[0])

    def sort_body(c, carry):
        for j in range(K // 16):
            s = srcv[c, pl.ds(j * 16, 16)]
            d = dstv[c, pl.ds(j * 16, 16)]
            b = jnp.right_shift(s, 8)
            rank = plsc.scan_count(b)[0] - conv
            base = plsc.load_gather(cnt, [b])
            plsc.addupdate_scatter(cnt, [b], onesi)
            slot = base + rank
            r = jnp.right_shift(slot, 7)
            col = jnp.bitwise_and(slot, 127)
            plsc.store_scatter(src2, [r, col], s)
            plsc.store_scatter(dst2, [r, col], d)
        return carry

    lax.fori_loop(0, CH, sort_body, 0)
    pltpu.sync_copy(src2.at[pl.ds(0, CH)], srcout.at[w])
    pltpu.sync_copy(dst2.at[pl.ds(0, CH)], dstout.at[w])


def _make_scatter(npass):
    IB = 8

    @functools.partial(
        pl.kernel,
        out_type=[jax.ShapeDtypeStruct((NC, NR, CP), jnp.float32)] * npass,
        mesh=_mesh,
        scratch_types=[
            pltpu.VMEM((3 * IB, K), jnp.int32),
            pltpu.VMEM((CH, K), jnp.int32),
            pltpu.VMEM((K, CP), jnp.float32),
            pltpu.VMEM((K, CP), jnp.float32),
            pltpu.VMEM_SHARED((NR, CP), jnp.float32),
            pltpu.SemaphoreType.DMA,
            pltpu.SemaphoreType.DMA,
            pltpu.SemaphoreType.DMA,
        ],
        compiler_params=pltpu.CompilerParams(needs_layout_passes=False,
                                             use_tc_tiling_on_sc=False),
    )
    def _scatter_kernel(*refs):
        zs = refs[:npass]
        srcw_hbm, dstw_hbm, zrow_hbm = refs[npass:npass + 3]
        outs = refs[npass + 3:2 * npass + 3]
        (src_st, dst_v, buf0, buf1, acc,
         sem0, sem1, isem) = refs[2 * npass + 3:]

        cid = lax.axis_index("c")
        sid = lax.axis_index("s")
        w = cid * NS + sid
        row0 = pl.multiple_of(sid * RPT, 8)
        pltpu.sync_copy(dstw_hbm.at[w], dst_v)

        def one_pass(z_hbm, out_hbm):
            pltpu.sync_copy(zrow_hbm, acc.at[pl.ds(row0, RPT)])
            pltpu.sync_copy(srcw_hbm.at[w, pl.ds(0, IB)],
                            src_st.at[pl.ds(0, IB)])
            pltpu.async_copy(srcw_hbm.at[w, pl.ds(IB, IB)],
                             src_st.at[pl.ds(IB, IB)], isem)
            plsc.subcore_barrier()

            pltpu.async_copy(z_hbm.at[src_st.at[0]], buf0, sem0)

            def body(c, carry):
                nxt = c + 1

                @pl.when(jnp.logical_and(nxt % IB == 0, nxt < CH))
                def _():
                    m = nxt // IB
                    pltpu.make_async_copy(
                        srcw_hbm.at[w, pl.ds(pl.multiple_of(nxt, IB), IB)],
                        src_st.at[pl.ds(pl.multiple_of((m % 3) * IB, IB), IB)],
                        isem).wait()

                    @pl.when(nxt + IB < CH)
                    def _():
                        pltpu.async_copy(
                            srcw_hbm.at[w, pl.ds(pl.multiple_of(nxt + IB, IB),
                                                 IB)],
                            src_st.at[pl.ds(
                                pl.multiple_of(((m + 1) % 3) * IB, IB), IB)],
                            isem)

                @pl.when(jnp.logical_and(nxt < CH, nxt % 2 == 0))
                def _():
                    pltpu.async_copy(z_hbm.at[src_st.at[nxt % (3 * IB)]],
                                     buf0, sem0)

                @pl.when(jnp.logical_and(nxt < CH, nxt % 2 == 1))
                def _():
                    pltpu.async_copy(z_hbm.at[src_st.at[nxt % (3 * IB)]],
                                     buf1, sem1)

                @pl.when(c % 2 == 0)
                def _():
                    pltpu.make_async_copy(z_hbm.at[src_st.at[c % (3 * IB)]],
                                          buf0, sem0).wait()
                    pltpu.sync_copy(buf0, acc.at[dst_v.at[c]], add=True)

                @pl.when(c % 2 == 1)
                def _():
                    pltpu.make_async_copy(z_hbm.at[src_st.at[c % (3 * IB)]],
                                          buf1, sem1).wait()
                    pltpu.sync_copy(buf1, acc.at[dst_v.at[c]], add=True)

                return carry

            lax.fori_loop(0, CH, body, 0)

            plsc.subcore_barrier()
            pltpu.sync_copy(acc.at[pl.ds(row0, RPT)],
                            out_hbm.at[cid, pl.ds(row0, RPT)])

        for p in range(npass):
            one_pass(zs[p], outs[p])
            if p + 1 < npass:
                plsc.subcore_barrier()

    return _scatter_kernel


_scatter2 = _make_scatter(2)
_scatter1 = _make_scatter(1)


_R = 1000
_G = N // _R


def _lrelu(v):
    return jnp.where(v >= 0, v, 0.01 * v)


def _zspec():
    return pl.BlockSpec((_R, CP), lambda i: (i, 0))


def _pspec():
    return pl.BlockSpec((2, _R, CP), lambda i: (0, i, 0))


def _tc_first(deg2, x, Wt):
    def body(deg_ref, x_ref, w_ref, dis_ref, zl_ref, zh_ref):
        deg = jnp.sum(deg_ref[...], axis=1, keepdims=True) + 1.0
        dis = lax.rsqrt(deg)
        dis_ref[...] = dis
        mm = dis * jnp.dot(x_ref[...], w_ref[...],
                           preferred_element_type=jnp.float32)
        zl_ref[...] = mm[:, :CP]
        zh_ref[...] = mm[:, CP:]

    return pl.pallas_call(
        body,
        grid=(_G,),
        in_specs=[
            pl.BlockSpec((_R, NW), lambda i: (i, 0)),
            pl.BlockSpec((_R, IN_DIM), lambda i: (i, 0)),
            pl.BlockSpec((IN_DIM, HID), lambda i: (0, 0)),
        ],
        out_specs=[pl.BlockSpec((_R, 1), lambda i: (i, 0)),
                   _zspec(), _zspec()],
        out_shape=[
            jax.ShapeDtypeStruct((N, 1), jnp.float32),
            jax.ShapeDtypeStruct((NR, CP), jnp.float32),
            jax.ShapeDtypeStruct((NR, CP), jnp.float32),
        ],
    )(deg2, x, Wt)


def _tc_stage(ps, zs, dis, b, Wt, res=None, mm_bias=None, mm_res=None, *,
              use_lrelu=True, scale_out=True, emit_h=False, split_out=True):
    nh = len(zs)
    d_in = nh * CP
    d_out = Wt.shape[1]
    nout = (d_out // CP) if (split_out and d_out % CP == 0) else 1

    def body(*refs):
        p_refs = refs[:nh]
        z_refs = refs[nh:2 * nh]
        dis_ref, b_ref, w_ref = refs[2 * nh:2 * nh + 3]
        rest = refs[2 * nh + 3:]
        i = 0
        res_ref = mmb_ref = mmr_ref = None
        if res is not None:
            res_ref = rest[i]; i += 1
        if mm_bias is not None:
            mmb_ref = rest[i]; i += 1
        if mm_res is not None:
            mmr_ref = rest[i]; i += 1
        outs = rest[i:]
        dis_v = dis_ref[...]
        h = jnp.concatenate(
            [p[0] + p[1] + z[...] for p, z in zip(p_refs, z_refs)], axis=1)
        h = dis_v * h + b_ref[...]
        if use_lrelu:
            h = _lrelu(h)
        if res_ref is not None:
            h = h + res_ref[...]
        if emit_h:
            outs[-1][...] = h
        mm = jnp.dot(h, w_ref[...], preferred_element_type=jnp.float32)
        if scale_out:
            mm = dis_v * mm
        if mmb_ref is not None:
            mm = mm + mmb_ref[...]
        if mmr_ref is not None:
            mm = mm + mmr_ref[...]
        if nout == 1:
            outs[0][...] = mm
        else:
            for j in range(nout):
                outs[j][...] = mm[:, j * CP:(j + 1) * CP]

    in_specs = ([_pspec() for _ in range(nh)] + [_zspec() for _ in range(nh)]
                + [pl.BlockSpec((_R, 1), lambda i: (i, 0)),
                   pl.BlockSpec((1, d_in), lambda i: (0, 0)),
                   pl.BlockSpec((d_in, d_out), lambda i: (0, 0))])
    args = list(ps) + list(zs) + [dis, b.reshape(1, d_in), Wt]
    if res is not None:
        in_specs.append(pl.BlockSpec((_R, d_in), lambda i: (i, 0)))
        args.append(res)
    if mm_bias is not None:
        in_specs.append(pl.BlockSpec((1, d_out), lambda i: (0, 0)))
        args.append(mm_bias.reshape(1, d_out))
    if mm_res is not None:
        in_specs.append(pl.BlockSpec((_R, d_out), lambda i: (i, 0)))
        args.append(mm_res)

    if nout == 1:
        out_specs = [pl.BlockSpec((_R, d_out), lambda i: (i, 0))]
        rows = NR if split_out and d_out == CP else N
        out_shape = [jax.ShapeDtypeStruct((rows, d_out), jnp.float32)]
    else:
        out_specs = [_zspec() for _ in range(nout)]
        out_shape = [jax.ShapeDtypeStruct((NR, CP), jnp.float32)] * nout
    if emit_h:
        out_specs.append(pl.BlockSpec((_R, d_in), lambda i: (i, 0)))
        out_shape.append(jax.ShapeDtypeStruct((N, d_in), jnp.float32))

    return pl.pallas_call(
        body,
        grid=(_G,),
        in_specs=in_specs,
        out_specs=out_specs,
        out_shape=out_shape,
    )(*args)


def kernel(x, edge_index, W1, b1, W2, b2, W3, b3, Wl, bl, Wd1, bd1, Wlin, blin):
    src = edge_index[0]
    dst = edge_index[1]
    srcw = jnp.pad(src.reshape(NW, EPW),
                   ((0, 0), (0, CH * K - EPW))).reshape(NW, CH, K)
    dstw = jnp.pad(dst.reshape(NW, EPW), ((0, 0), (0, CH * K - EPW)),
                   constant_values=N).reshape(NW, CH, K)
    zdeg = jnp.zeros((NR,), jnp.float32)
    zrow = jnp.zeros((RPT, CP), jnp.float32)

    degp, srcw2, dstw2 = _deg_kernel(srcw, dstw, zdeg)
    deg2 = degp.T

    dis, z1l, z1h = _tc_first(deg2, x, W1.T)
    p1l, p1h = _scatter2(z1l, z1h, srcw2, dstw2, zrow)
    z2l, z2h = _tc_stage((p1l, p1h), (z1l, z1h), dis, b1, W2.T)
    p2l, p2h = _scatter2(z2l, z2h, srcw2, dstw2, zrow)
    z3l, z3h, x2 = _tc_stage((p2l, p2h), (z2l, z2h), dis, b2, W3.T,
                             emit_h=True)
    p3l, p3h = _scatter2(z3l, z3h, srcw2, dstw2, zrow)
    z4 = _tc_stage((p3l, p3h), (z3l, z3h), dis, b3, Wl.T, res=x2)[0]
    p4 = _scatter1(z4, srcw2, dstw2, zrow)[0]
    z5l, z5h, zlat = _tc_stage((p4,), (z4,), dis, bl, Wd1.T,
                               use_lrelu=False, emit_h=True)
    p5l, p5h = _scatter2(z5l, z5h, srcw2, dstw2, zrow)
    rec = _tc_stage((p5l, p5h), (z5l, z5h), dis, bd1, Wlin.T,
                    mm_bias=blin, mm_res=x, scale_out=False,
                    split_out=False)[0]
    return (rec, zlat)


# speedup vs baseline: 1.3218x; 1.3218x over previous
import functools

import jax
import jax.numpy as jnp
from jax import lax
from jax.experimental import pallas as pl
from jax.experimental.pallas import tpu as pltpu
from jax.experimental.pallas import tpu_sc as plsc

N = 10000
E = 320000
IN_DIM = 128
HID = 128
LAT = 64
CP = 64
CW = CP // 2

NC = 2
NS = 16
NW = NC * NS
EPW = E // NW
K = 128
CH = EPW // K if EPW % K == 0 else EPW // K + 1
CH = CH + (CH % 2)
NR = ((N + 1) + NS * 8 - 1) // (NS * 8) * (NS * 8)
RPT = NR // NS

_mesh = plsc.VectorSubcoreMesh(core_axis_name="c", subcore_axis_name="s")


@functools.partial(
    pl.kernel,
    out_type=jax.ShapeDtypeStruct((NW, NR), jnp.float32),
    mesh=_mesh,
    scratch_types=[
        pltpu.VMEM((CH, K), jnp.int32),
        pltpu.VMEM((NR,), jnp.float32),
    ],
    compiler_params=pltpu.CompilerParams(needs_layout_passes=False),
)
def _deg_kernel(dstw_hbm, zdeg_hbm, out_hbm, idx_v, degv):
    w = lax.axis_index("c") * NS + lax.axis_index("s")
    pltpu.sync_copy(dstw_hbm.at[w], idx_v)
    pltpu.sync_copy(zdeg_hbm, degv)
    ones = jnp.ones((16,), jnp.float32)

    def body(c, carry):
        for j in range(K // 16):
            idx = idx_v[c, pl.ds(j * 16, 16)]
            plsc.addupdate_scatter(degv, [idx], ones)
        return carry

    lax.fori_loop(0, CH, body, 0)
    pltpu.sync_copy(degv, out_hbm.at[w])


def _make_scatter(npass):
    IB = 8

    @functools.partial(
        pl.kernel,
        out_type=[jax.ShapeDtypeStruct((NC, NR, CP), jnp.float32)] * npass,
        mesh=_mesh,
        scratch_types=[
            pltpu.VMEM((3 * IB, K), jnp.int32),
            pltpu.VMEM((CH, K), jnp.int32),
            pltpu.VMEM((K, CW), jnp.int32),
            pltpu.VMEM((K, CW), jnp.int32),
            pltpu.VMEM((K, CP), jnp.float32),
            pltpu.VMEM((K, CP), jnp.float32),
            pltpu.VMEM_SHARED((NR, CP), jnp.float32),
            pltpu.SemaphoreType.DMA,
            pltpu.SemaphoreType.DMA,
            pltpu.SemaphoreType.DMA,
            pltpu.SemaphoreType.DMA,
            pltpu.SemaphoreType.DMA,
        ],
        compiler_params=pltpu.CompilerParams(needs_layout_passes=False,
                                             use_tc_tiling_on_sc=False),
    )
    def _scatter_kernel(*refs):
        zs = refs[:npass]
        srcw_hbm, dstw_hbm, zrow_hbm = refs[npass:npass + 3]
        outs = refs[npass + 3:2 * npass + 3]
        (src_st, dst_v, bufb0, bufb1, buff0, buff1, acc,
         sem0, sem1, ssem0, ssem1, isem) = refs[2 * npass + 3:]

        cid = lax.axis_index("c")
        sid = lax.axis_index("s")
        w = cid * NS + sid
        row0 = pl.multiple_of(sid * RPT, 8)
        pltpu.sync_copy(dstw_hbm.at[w], dst_v)
        mask_hi = jnp.full((16,), -65536, jnp.int32)

        def unpack(bufb, buff):
            def conv(r, carry):
                w0 = bufb[r, pl.ds(0, 16)]
                w1 = bufb[r, pl.ds(16, 16)]
                buff[r, pl.ds(0, 16)] = plsc.bitcast(
                    lax.shift_left(w0, 16), jnp.float32)
                buff[r, pl.ds(16, 16)] = plsc.bitcast(
                    jnp.bitwise_and(w0, mask_hi), jnp.float32)
                buff[r, pl.ds(32, 16)] = plsc.bitcast(
                    lax.shift_left(w1, 16), jnp.float32)
                buff[r, pl.ds(48, 16)] = plsc.bitcast(
                    jnp.bitwise_and(w1, mask_hi), jnp.float32)
                return carry

            lax.fori_loop(0, K, conv, 0)

        def one_pass(z_hbm, out_hbm):
            pltpu.sync_copy(zrow_hbm, acc.at[pl.ds(row0, RPT)])
            pltpu.sync_copy(srcw_hbm.at[w, pl.ds(0, IB)],
                            src_st.at[pl.ds(0, IB)])
            pltpu.async_copy(srcw_hbm.at[w, pl.ds(IB, IB)],
                             src_st.at[pl.ds(IB, IB)], isem)
            plsc.subcore_barrier()

            pltpu.async_copy(z_hbm.at[src_st.at[0]], bufb0, sem0)

            def body(c, carry):
                nxt = c + 1

                @pl.when(jnp.logical_and(nxt % IB == 0, nxt < CH))
                def _():
                    m = nxt // IB
                    pltpu.make_async_copy(
                        srcw_hbm.at[w, pl.ds(pl.multiple_of(nxt, IB), IB)],
                        src_st.at[pl.ds(pl.multiple_of((m % 3) * IB, IB), IB)],
                        isem).wait()

                    @pl.when(nxt + IB < CH)
                    def _():
                        pltpu.async_copy(
                            srcw_hbm.at[w, pl.ds(pl.multiple_of(nxt + IB, IB),
                                                 IB)],
                            src_st.at[pl.ds(
                                pl.multiple_of(((m + 1) % 3) * IB, IB), IB)],
                            isem)

                @pl.when(jnp.logical_and(nxt < CH, nxt % 2 == 0))
                def _():
                    pltpu.async_copy(z_hbm.at[src_st.at[nxt % (3 * IB)]],
                                     bufb0, sem0)

                @pl.when(jnp.logical_and(nxt < CH, nxt % 2 == 1))
                def _():
                    pltpu.async_copy(z_hbm.at[src_st.at[nxt % (3 * IB)]],
                                     bufb1, sem1)

                @pl.when(c % 2 == 0)
                def _():
                    pltpu.make_async_copy(z_hbm.at[src_st.at[c % (3 * IB)]],
                                          bufb0, sem0).wait()

                    @pl.when(c >= 2)
                    def _():
                        pltpu.make_async_copy(
                            buff0, acc.at[dst_v.at[c - 2]], ssem0).wait()

                    unpack(bufb0, buff0)
                    pltpu.async_copy(buff0, acc.at[dst_v.at[c]], ssem0,
                                     add=True)

                @pl.when(c % 2 == 1)
                def _():
                    pltpu.make_async_copy(z_hbm.at[src_st.at[c % (3 * IB)]],
                                          bufb1, sem1).wait()

                    @pl.when(c >= 2)
                    def _():
                        pltpu.make_async_copy(
                            buff1, acc.at[dst_v.at[c - 2]], ssem1).wait()

                    unpack(bufb1, buff1)
                    pltpu.async_copy(buff1, acc.at[dst_v.at[c]], ssem1,
                                     add=True)

                return carry

            lax.fori_loop(0, CH, body, 0)
            pltpu.make_async_copy(buff0, acc.at[dst_v.at[CH - 2]],
                                  ssem0).wait()
            pltpu.make_async_copy(buff1, acc.at[dst_v.at[CH - 1]],
                                  ssem1).wait()

            plsc.subcore_barrier()
            pltpu.sync_copy(acc.at[pl.ds(row0, RPT)],
                            out_hbm.at[cid, pl.ds(row0, RPT)])

        for p in range(npass):
            one_pass(zs[p], outs[p])
            if p + 1 < npass:
                plsc.subcore_barrier()

    return _scatter_kernel


_scatter2 = _make_scatter(2)
_scatter1 = _make_scatter(1)


_R = 1000
_G = N // _R


def _lrelu(v):
    return jnp.where(v >= 0, v, 0.01 * v)


def _b16i(v):
    return lax.bitcast_convert_type(
        v.astype(jnp.bfloat16).astype(jnp.float32), jnp.int32)


def _pack_half(t):
    parts = []
    for g in (0, 32):
        a = _b16i(t[:, g:g + 16])
        b = _b16i(t[:, g + 16:g + 32])
        parts.append(jnp.bitwise_or(lax.shift_right_logical(a, 16), b))
    return jnp.concatenate(parts, axis=1)


def _zspec():
    return pl.BlockSpec((_R, CP), lambda i: (i, 0))


def _zbspec():
    return pl.BlockSpec((_R, CW), lambda i: (i, 0))


def _pspec():
    return pl.BlockSpec((2, _R, CP), lambda i: (0, i, 0))


def _zshapes(nh):
    return ([jax.ShapeDtypeStruct((NR, CP), jnp.float32)] * nh
            + [jax.ShapeDtypeStruct((NR, CW), jnp.int32)] * nh)


def _tc_first(deg2, x, Wt):
    def body(deg_ref, x_ref, w_ref, dis_ref, zl_ref, zh_ref, bl_ref, bh_ref):
        deg = jnp.sum(deg_ref[...], axis=1, keepdims=True) + 1.0
        dis = lax.rsqrt(deg)
        dis_ref[...] = dis
        mm = dis * jnp.dot(x_ref[...], w_ref[...],
                           preferred_element_type=jnp.float32)
        zl_ref[...] = mm[:, :CP]
        zh_ref[...] = mm[:, CP:]
        bl_ref[...] = _pack_half(mm[:, :CP])
        bh_ref[...] = _pack_half(mm[:, CP:])

    return pl.pallas_call(
        body,
        grid=(_G,),
        in_specs=[
            pl.BlockSpec((_R, NW), lambda i: (i, 0)),
            pl.BlockSpec((_R, IN_DIM), lambda i: (i, 0)),
            pl.BlockSpec((IN_DIM, HID), lambda i: (0, 0)),
        ],
        out_specs=[pl.BlockSpec((_R, 1), lambda i: (i, 0)),
                   _zspec(), _zspec(), _zbspec(), _zbspec()],
        out_shape=[jax.ShapeDtypeStruct((N, 1), jnp.float32)] + _zshapes(2),
    )(deg2, x, Wt)


def _tc_stage(ps, zs, dis, b, Wt, res=None, mm_bias=None, mm_res=None, *,
              use_lrelu=True, scale_out=True, emit_h=False, split_out=True):
    nh = len(zs)
    d_in = nh * CP
    d_out = Wt.shape[1]
    nout = (d_out // CP) if (split_out and d_out % CP == 0) else 1

    def body(*refs):
        p_refs = refs[:nh]
        z_refs = refs[nh:2 * nh]
        dis_ref, b_ref, w_ref = refs[2 * nh:2 * nh + 3]
        rest = refs[2 * nh + 3:]
        i = 0
        res_ref = mmb_ref = mmr_ref = None
        if res is not None:
            res_ref = rest[i]; i += 1
        if mm_bias is not None:
            mmb_ref = rest[i]; i += 1
        if mm_res is not None:
            mmr_ref = rest[i]; i += 1
        outs = rest[i:]
        dis_v = dis_ref[...]
        h = jnp.concatenate(
            [p[0] + p[1] + z[...] for p, z in zip(p_refs, z_refs)], axis=1)
        h = dis_v * h + b_ref[...]
        if use_lrelu:
            h = _lrelu(h)
        if res_ref is not None:
            h = h + res_ref[...]
        if emit_h:
            outs[-1][...] = h
        mm = jnp.dot(h, w_ref[...], preferred_element_type=jnp.float32)
        if scale_out:
            mm = dis_v * mm
        if mmb_ref is not None:
            mm = mm + mmb_ref[...]
        if mmr_ref is not None:
            mm = mm + mmr_ref[...]
        if not split_out:
            outs[0][...] = mm
        else:
            for j in range(nout):
                t = mm[:, j * CP:(j + 1) * CP]
                outs[j][...] = t
                outs[nout + j][...] = _pack_half(t)

    in_specs = ([_pspec() for _ in range(nh)] + [_zspec() for _ in range(nh)]
                + [pl.BlockSpec((_R, 1), lambda i: (i, 0)),
                   pl.BlockSpec((1, d_in), lambda i: (0, 0)),
                   pl.BlockSpec((d_in, d_out), lambda i: (0, 0))])
    args = list(ps) + list(zs) + [dis, b.reshape(1, d_in), Wt]
    if res is not None:
        in_specs.append(pl.BlockSpec((_R, d_in), lambda i: (i, 0)))
        args.append(res)
    if mm_bias is not None:
        in_specs.append(pl.BlockSpec((1, d_out), lambda i: (0, 0)))
        args.append(mm_bias.reshape(1, d_out))
    if mm_res is not None:
        in_specs.append(pl.BlockSpec((_R, d_out), lambda i: (i, 0)))
        args.append(mm_res)

    if not split_out:
        out_specs = [pl.BlockSpec((_R, d_out), lambda i: (i, 0))]
        out_shape = [jax.ShapeDtypeStruct((N, d_out), jnp.float32)]
    else:
        out_specs = ([_zspec() for _ in range(nout)]
                     + [_zbspec() for _ in range(nout)])
        out_shape = _zshapes(nout)
    if emit_h:
        out_specs.append(pl.BlockSpec((_R, d_in), lambda i: (i, 0)))
        out_shape.append(jax.ShapeDtypeStruct((N, d_in), jnp.float32))

    return pl.pallas_call(
        body,
        grid=(_G,),
        in_specs=in_specs,
        out_specs=out_specs,
        out_shape=out_shape,
    )(*args)


def kernel(x, edge_index, W1, b1, W2, b2, W3, b3, Wl, bl, Wd1, bd1, Wlin, blin):
    src = edge_index[0]
    dst = edge_index[1]
    srcw = jnp.pad(src.reshape(NW, EPW),
                   ((0, 0), (0, CH * K - EPW))).reshape(NW, CH, K)
    dstw = jnp.pad(dst.reshape(NW, EPW), ((0, 0), (0, CH * K - EPW)),
                   constant_values=N).reshape(NW, CH, K)
    zdeg = jnp.zeros((NR,), jnp.float32)
    zrow = jnp.zeros((RPT, CP), jnp.float32)

    degp = _deg_kernel(dstw, zdeg)
    deg2 = degp.T

    dis, z1l, z1h, zb1l, zb1h = _tc_first(deg2, x, W1.T)
    p1l, p1h = _scatter2(zb1l, zb1h, srcw, dstw, zrow)
    z2l, z2h, zb2l, zb2h = _tc_stage((p1l, p1h), (z1l, z1h), dis, b1, W2.T)
    p2l, p2h = _scatter2(zb2l, zb2h, srcw, dstw, zrow)
    z3l, z3h, zb3l, zb3h, x2 = _tc_stage((p2l, p2h), (z2l, z2h), dis, b2,
                                         W3.T, emit_h=True)
    p3l, p3h = _scatter2(zb3l, zb3h, srcw, dstw, zrow)
    z4, zb4 = _tc_stage((p3l, p3h), (z3l, z3h), dis, b3, Wl.T, res=x2)
    p4 = _scatter1(zb4, srcw, dstw, zrow)[0]
    z5l, z5h, zb5l, zb5h, zlat = _tc_stage((p4,), (z4,), dis, bl, Wd1.T,
                                           use_lrelu=False, emit_h=True)
    p5l, p5h = _scatter2(zb5l, zb5h, srcw, dstw, zrow)
    rec = _tc_stage((p5l, p5h), (z5l, z5h), dis, bd1, Wlin.T,
                    mm_bias=blin, mm_res=x, scale_out=False,
                    split_out=False)[0]
    return (rec, zlat)


# speedup vs baseline: 1.5935x; 1.2056x over previous
import functools

import jax
import jax.numpy as jnp
from jax import lax
from jax.experimental import pallas as pl
from jax.experimental.pallas import tpu as pltpu
from jax.experimental.pallas import tpu_sc as plsc

N = 10000
E = 320000
IN_DIM = 128
HID = 128
LAT = 64
CP = 64
CW = CP // 2

NC = 2
NS = 16
NW = NC * NS
EPW = E // NW
K = 128
CH = EPW // K if EPW % K == 0 else EPW // K + 1
CH = CH + (CH % 2)
NR = ((N + 1) + NS * 8 - 1) // (NS * 8) * (NS * 8)
RPT = NR // NS

_mesh = plsc.VectorSubcoreMesh(core_axis_name="c", subcore_axis_name="s")


@functools.partial(
    pl.kernel,
    out_type=jax.ShapeDtypeStruct((NW, NR), jnp.float32),
    mesh=_mesh,
    scratch_types=[
        pltpu.VMEM((CH, K), jnp.int32),
        pltpu.VMEM((NR,), jnp.float32),
    ],
    compiler_params=pltpu.CompilerParams(needs_layout_passes=False),
)
def _deg_kernel(dstw_hbm, zdeg_hbm, out_hbm, idx_v, degv):
    w = lax.axis_index("c") * NS + lax.axis_index("s")
    pltpu.sync_copy(dstw_hbm.at[w], idx_v)
    pltpu.sync_copy(zdeg_hbm, degv)
    ones = jnp.ones((16,), jnp.float32)

    def body(c, carry):
        for j in range(K // 16):
            idx = idx_v[c, pl.ds(j * 16, 16)]
            plsc.addupdate_scatter(degv, [idx], ones)
        return carry

    lax.fori_loop(0, CH, body, 0)
    pltpu.sync_copy(degv, out_hbm.at[w])


def _make_scatter(npass):
    IB = 8

    @functools.partial(
        pl.kernel,
        out_type=[jax.ShapeDtypeStruct((NC, NR, CP), jnp.float32)] * npass,
        mesh=_mesh,
        scratch_types=[
            pltpu.VMEM((3 * IB, K), jnp.int32),
            pltpu.VMEM((CH, K), jnp.int32),
            pltpu.VMEM((K, CW), jnp.int32),
            pltpu.VMEM((K, CW), jnp.int32),
            pltpu.VMEM((K, CP), jnp.float32),
            pltpu.VMEM((K, CP), jnp.float32),
            pltpu.VMEM_SHARED((NR, CP), jnp.float32),
            pltpu.SemaphoreType.DMA,
            pltpu.SemaphoreType.DMA,
            pltpu.SemaphoreType.DMA,
            pltpu.SemaphoreType.DMA,
            pltpu.SemaphoreType.DMA,
        ],
        compiler_params=pltpu.CompilerParams(needs_layout_passes=False,
                                             use_tc_tiling_on_sc=False),
    )
    def _scatter_kernel(*refs):
        zs = refs[:npass]
        srcw_hbm, dstw_hbm, zrow_hbm = refs[npass:npass + 3]
        outs = refs[npass + 3:2 * npass + 3]
        (src_st, dst_v, bufb0, bufb1, buff0, buff1, acc,
         sem0, sem1, ssem0, ssem1, isem) = refs[2 * npass + 3:]

        cid = lax.axis_index("c")
        sid = lax.axis_index("s")
        w = cid * NS + sid
        row0 = pl.multiple_of(sid * RPT, 8)
        pltpu.sync_copy(dstw_hbm.at[w], dst_v)
        mask_hi = jnp.full((16,), -65536, jnp.int32)

        def unpack(bufb, buff):
            @plsc.parallel_loop(0, K, step=1, unroll=4)
            def _(r):
                w0 = bufb[r, pl.ds(0, 16)]
                w1 = bufb[r, pl.ds(16, 16)]
                buff[r, pl.ds(0, 16)] = plsc.bitcast(
                    lax.shift_left(w0, 16), jnp.float32)
                buff[r, pl.ds(16, 16)] = plsc.bitcast(
                    jnp.bitwise_and(w0, mask_hi), jnp.float32)
                buff[r, pl.ds(32, 16)] = plsc.bitcast(
                    lax.shift_left(w1, 16), jnp.float32)
                buff[r, pl.ds(48, 16)] = plsc.bitcast(
                    jnp.bitwise_and(w1, mask_hi), jnp.float32)

        def one_pass(z_hbm, out_hbm):
            pltpu.sync_copy(zrow_hbm, acc.at[pl.ds(row0, RPT)])
            pltpu.sync_copy(srcw_hbm.at[w, pl.ds(0, IB)],
                            src_st.at[pl.ds(0, IB)])
            pltpu.async_copy(srcw_hbm.at[w, pl.ds(IB, IB)],
                             src_st.at[pl.ds(IB, IB)], isem)
            plsc.subcore_barrier()

            pltpu.async_copy(z_hbm.at[src_st.at[0]], bufb0, sem0)

            def body(c, carry):
                nxt = c + 1

                @pl.when(jnp.logical_and(nxt % IB == 0, nxt < CH))
                def _():
                    m = nxt // IB
                    pltpu.make_async_copy(
                        srcw_hbm.at[w, pl.ds(pl.multiple_of(nxt, IB), IB)],
                        src_st.at[pl.ds(pl.multiple_of((m % 3) * IB, IB), IB)],
                        isem).wait()

                    @pl.when(nxt + IB < CH)
                    def _():
                        pltpu.async_copy(
                            srcw_hbm.at[w, pl.ds(pl.multiple_of(nxt + IB, IB),
                                                 IB)],
                            src_st.at[pl.ds(
                                pl.multiple_of(((m + 1) % 3) * IB, IB), IB)],
                            isem)

                @pl.when(jnp.logical_and(nxt < CH, nxt % 2 == 0))
                def _():
                    pltpu.async_copy(z_hbm.at[src_st.at[nxt % (3 * IB)]],
                                     bufb0, sem0)

                @pl.when(jnp.logical_and(nxt < CH, nxt % 2 == 1))
                def _():
                    pltpu.async_copy(z_hbm.at[src_st.at[nxt % (3 * IB)]],
                                     bufb1, sem1)

                @pl.when(c % 2 == 0)
                def _():
                    pltpu.make_async_copy(z_hbm.at[src_st.at[c % (3 * IB)]],
                                          bufb0, sem0).wait()

                    @pl.when(c >= 2)
                    def _():
                        pltpu.make_async_copy(
                            buff0, acc.at[dst_v.at[c - 2]], ssem0).wait()

                    unpack(bufb0, buff0)
                    pltpu.async_copy(buff0, acc.at[dst_v.at[c]], ssem0,
                                     add=True)

                @pl.when(c % 2 == 1)
                def _():
                    pltpu.make_async_copy(z_hbm.at[src_st.at[c % (3 * IB)]],
                                          bufb1, sem1).wait()

                    @pl.when(c >= 2)
                    def _():
                        pltpu.make_async_copy(
                            buff1, acc.at[dst_v.at[c - 2]], ssem1).wait()

                    unpack(bufb1, buff1)
                    pltpu.async_copy(buff1, acc.at[dst_v.at[c]], ssem1,
                                     add=True)

                return carry

            lax.fori_loop(0, CH, body, 0)
            pltpu.make_async_copy(buff0, acc.at[dst_v.at[CH - 2]],
                                  ssem0).wait()
            pltpu.make_async_copy(buff1, acc.at[dst_v.at[CH - 1]],
                                  ssem1).wait()

            plsc.subcore_barrier()
            pltpu.sync_copy(acc.at[pl.ds(row0, RPT)],
                            out_hbm.at[cid, pl.ds(row0, RPT)])

        for p in range(npass):
            one_pass(zs[p], outs[p])
            if p + 1 < npass:
                plsc.subcore_barrier()

    return _scatter_kernel


_scatter2 = _make_scatter(2)
_scatter1 = _make_scatter(1)


_R = 1000
_G = N // _R


def _lrelu(v):
    return jnp.where(v >= 0, v, 0.01 * v)


def _b16i(v):
    return lax.bitcast_convert_type(
        v.astype(jnp.bfloat16).astype(jnp.float32), jnp.int32)


def _pack_half(t):
    parts = []
    for g in (0, 32):
        a = _b16i(t[:, g:g + 16])
        b = _b16i(t[:, g + 16:g + 32])
        parts.append(jnp.bitwise_or(lax.shift_right_logical(a, 16), b))
    return jnp.concatenate(parts, axis=1)


def _zspec():
    return pl.BlockSpec((_R, CP), lambda i: (i, 0))


def _zbspec():
    return pl.BlockSpec((_R, CW), lambda i: (i, 0))


def _pspec():
    return pl.BlockSpec((2, _R, CP), lambda i: (0, i, 0))


def _zshapes(nh):
    return ([jax.ShapeDtypeStruct((NR, CP), jnp.float32)] * nh
            + [jax.ShapeDtypeStruct((NR, CW), jnp.int32)] * nh)


def _tc_first(deg2, x, Wt):
    def body(deg_ref, x_ref, w_ref, dis_ref, zl_ref, zh_ref, bl_ref, bh_ref):
        deg = jnp.sum(deg_ref[...], axis=1, keepdims=True) + 1.0
        dis = lax.rsqrt(deg)
        dis_ref[...] = dis
        mm = dis * jnp.dot(x_ref[...], w_ref[...],
                           preferred_element_type=jnp.float32)
        zl_ref[...] = mm[:, :CP]
        zh_ref[...] = mm[:, CP:]
        bl_ref[...] = _pack_half(mm[:, :CP])
        bh_ref[...] = _pack_half(mm[:, CP:])

    return pl.pallas_call(
        body,
        grid=(_G,),
        in_specs=[
            pl.BlockSpec((_R, NW), lambda i: (i, 0)),
            pl.BlockSpec((_R, IN_DIM), lambda i: (i, 0)),
            pl.BlockSpec((IN_DIM, HID), lambda i: (0, 0)),
        ],
        out_specs=[pl.BlockSpec((_R, 1), lambda i: (i, 0)),
                   _zspec(), _zspec(), _zbspec(), _zbspec()],
        out_shape=[jax.ShapeDtypeStruct((N, 1), jnp.float32)] + _zshapes(2),
    )(deg2, x, Wt)


def _tc_stage(ps, zs, dis, b, Wt, res=None, mm_bias=None, mm_res=None, *,
              use_lrelu=True, scale_out=True, emit_h=False, split_out=True):
    nh = len(zs)
    d_in = nh * CP
    d_out = Wt.shape[1]
    nout = (d_out // CP) if (split_out and d_out % CP == 0) else 1

    def body(*refs):
        p_refs = refs[:nh]
        z_refs = refs[nh:2 * nh]
        dis_ref, b_ref, w_ref = refs[2 * nh:2 * nh + 3]
        rest = refs[2 * nh + 3:]
        i = 0
        res_ref = mmb_ref = mmr_ref = None
        if res is not None:
            res_ref = rest[i]; i += 1
        if mm_bias is not None:
            mmb_ref = rest[i]; i += 1
        if mm_res is not None:
            mmr_ref = rest[i]; i += 1
        outs = rest[i:]
        dis_v = dis_ref[...]
        h = jnp.concatenate(
            [p[0] + p[1] + z[...] for p, z in zip(p_refs, z_refs)], axis=1)
        h = dis_v * h + b_ref[...]
        if use_lrelu:
            h = _lrelu(h)
        if res_ref is not None:
            h = h + res_ref[...]
        if emit_h:
            outs[-1][...] = h
        mm = jnp.dot(h, w_ref[...], preferred_element_type=jnp.float32)
        if scale_out:
            mm = dis_v * mm
        if mmb_ref is not None:
            mm = mm + mmb_ref[...]
        if mmr_ref is not None:
            mm = mm + mmr_ref[...]
        if not split_out:
            outs[0][...] = mm
        else:
            for j in range(nout):
                t = mm[:, j * CP:(j + 1) * CP]
                outs[j][...] = t
                outs[nout + j][...] = _pack_half(t)

    in_specs = ([_pspec() for _ in range(nh)] + [_zspec() for _ in range(nh)]
                + [pl.BlockSpec((_R, 1), lambda i: (i, 0)),
                   pl.BlockSpec((1, d_in), lambda i: (0, 0)),
                   pl.BlockSpec((d_in, d_out), lambda i: (0, 0))])
    args = list(ps) + list(zs) + [dis, b.reshape(1, d_in), Wt]
    if res is not None:
        in_specs.append(pl.BlockSpec((_R, d_in), lambda i: (i, 0)))
        args.append(res)
    if mm_bias is not None:
        in_specs.append(pl.BlockSpec((1, d_out), lambda i: (0, 0)))
        args.append(mm_bias.reshape(1, d_out))
    if mm_res is not None:
        in_specs.append(pl.BlockSpec((_R, d_out), lambda i: (i, 0)))
        args.append(mm_res)

    if not split_out:
        out_specs = [pl.BlockSpec((_R, d_out), lambda i: (i, 0))]
        out_shape = [jax.ShapeDtypeStruct((N, d_out), jnp.float32)]
    else:
        out_specs = ([_zspec() for _ in range(nout)]
                     + [_zbspec() for _ in range(nout)])
        out_shape = _zshapes(nout)
    if emit_h:
        out_specs.append(pl.BlockSpec((_R, d_in), lambda i: (i, 0)))
        out_shape.append(jax.ShapeDtypeStruct((N, d_in), jnp.float32))

    return pl.pallas_call(
        body,
        grid=(_G,),
        in_specs=in_specs,
        out_specs=out_specs,
        out_shape=out_shape,
    )(*args)


def kernel(x, edge_index, W1, b1, W2, b2, W3, b3, Wl, bl, Wd1, bd1, Wlin, blin):
    src = edge_index[0]
    dst = edge_index[1]
    srcw = jnp.pad(src.reshape(NW, EPW),
                   ((0, 0), (0, CH * K - EPW))).reshape(NW, CH, K)
    dstw = jnp.pad(dst.reshape(NW, EPW), ((0, 0), (0, CH * K - EPW)),
                   constant_values=N).reshape(NW, CH, K)
    zdeg = jnp.zeros((NR,), jnp.float32)
    zrow = jnp.zeros((RPT, CP), jnp.float32)

    degp = _deg_kernel(dstw, zdeg)
    deg2 = degp.T

    dis, z1l, z1h, zb1l, zb1h = _tc_first(deg2, x, W1.T)
    p1l, p1h = _scatter2(zb1l, zb1h, srcw, dstw, zrow)
    z2l, z2h, zb2l, zb2h = _tc_stage((p1l, p1h), (z1l, z1h), dis, b1, W2.T)
    p2l, p2h = _scatter2(zb2l, zb2h, srcw, dstw, zrow)
    z3l, z3h, zb3l, zb3h, x2 = _tc_stage((p2l, p2h), (z2l, z2h), dis, b2,
                                         W3.T, emit_h=True)
    p3l, p3h = _scatter2(zb3l, zb3h, srcw, dstw, zrow)
    z4, zb4 = _tc_stage((p3l, p3h), (z3l, z3h), dis, b3, Wl.T, res=x2)
    p4 = _scatter1(zb4, srcw, dstw, zrow)[0]
    z5l, z5h, zb5l, zb5h, zlat = _tc_stage((p4,), (z4,), dis, bl, Wd1.T,
                                           use_lrelu=False, emit_h=True)
    p5l, p5h = _scatter2(zb5l, zb5h, srcw, dstw, zrow)
    rec = _tc_stage((p5l, p5h), (z5l, z5h), dis, bd1, Wlin.T,
                    mm_bias=blin, mm_res=x, scale_out=False,
                    split_out=False)[0]
    return (rec, zlat)
